# baseline (speedup 1.0000x reference)
import functools

import jax
import jax.numpy as jnp
from jax import lax
from jax.experimental import pallas as pl
from jax.experimental.pallas import tpu as pltpu
from jax.experimental.pallas import tpu_sc as plsc

BATCH = 16384
DIM = 32
PACK = 4
PDIM = DIM * PACK
NC = 2
NS = 16
L = 16
NW = NC * NS
BPW = BATCH // NW
CHUNK = 128
NCHUNK = BPW // CHUNK
NBUF = 3


def _sc_body(didx_hbm, aidx_hbm, dtab_hbm, atab_hbm, out_hbm,
             dpidx_v, apidx_v, doff_v, aoff_v, drows_v, arows_v, out_v,
             idx_tmp_v, sems):
    wid = lax.axis_index("s") * NC + lax.axis_index("c")
    base = wid * BPW

    pltpu.sync_copy(didx_hbm.at[pl.ds(base, BPW)], idx_tmp_v.at[0])
    pltpu.sync_copy(aidx_hbm.at[pl.ds(base, BPW)], idx_tmp_v.at[1])

    def prep(i, carry):
        s = pl.ds(i * L, L)
        dv = idx_tmp_v[0, s]
        av = idx_tmp_v[1, s]
        dpidx_v[s] = lax.shift_right_logical(dv, 2)
        apidx_v[s] = lax.shift_right_logical(av, 2)
        doff_v[s] = lax.shift_left(jnp.bitwise_and(dv, 3), 5)
        aoff_v[s] = lax.shift_left(jnp.bitwise_and(av, 3), 5)
        return carry

    lax.fori_loop(0, BPW // L, prep, 0)

    lane = lax.iota(jnp.int32, L)

    def start(c):
        b = c % NBUF
        sl = pl.ds(c * CHUNK, CHUNK)
        return (
            pltpu.async_copy(dtab_hbm.at[dpidx_v.at[sl]], drows_v.at[b],
                             sems.at[b]),
            pltpu.async_copy(atab_hbm.at[apidx_v.at[sl]], arows_v.at[b],
                             sems.at[b]),
        )

    inflight = {c: start(c) for c in range(min(NBUF, NCHUNK))}

    for c in range(NCHUNK):
        b = c % NBUF
        for cp in inflight.pop(c):
            cp.wait()

        dbuf = drows_v.at[b]
        abuf = arows_v.at[b]

        def group(g, carry):
            p0 = c * CHUNK + g * L
            rows = g * L + lane
            dcol = doff_v[pl.ds(p0, L)]
            acol = aoff_v[pl.ds(p0, L)]
            acc = jnp.zeros((L,), jnp.float32)
            for j in range(DIM):
                dv = plsc.load_gather(dbuf, [rows, dcol + j])
                av = plsc.load_gather(abuf, [rows, acol + j])
                acc = acc + dv * av
            out_v[pl.ds(p0, L)] = acc
            return carry

        lax.fori_loop(0, CHUNK // L, group, 0)

        if c + NBUF < NCHUNK:
            inflight[c + NBUF] = start(c + NBUF)

    pltpu.sync_copy(out_v, out_hbm.at[pl.ds(base, BPW)])


@functools.partial(
    pl.kernel,
    mesh=plsc.VectorSubcoreMesh(core_axis_name="c", subcore_axis_name="s"),
    out_type=jax.ShapeDtypeStruct((BATCH,), jnp.float32),
    scratch_types=[
        pltpu.VMEM((BPW,), jnp.int32),
        pltpu.VMEM((BPW,), jnp.int32),
        pltpu.VMEM((BPW,), jnp.int32),
        pltpu.VMEM((BPW,), jnp.int32),
        pltpu.VMEM((NBUF, CHUNK, PDIM), jnp.float32),
        pltpu.VMEM((NBUF, CHUNK, PDIM), jnp.float32),
        pltpu.VMEM((BPW,), jnp.float32),
        pltpu.VMEM((2, BPW), jnp.int32),
        pltpu.SemaphoreType.DMA((NBUF,)),
    ],
    compiler_params=pltpu.CompilerParams(needs_layout_passes=False),
)
def _sc_call(didx_hbm, aidx_hbm, dtab_hbm, atab_hbm, out_hbm,
             dpidx_v, apidx_v, doff_v, aoff_v, drows_v, arows_v, out_v,
             idx_tmp_v, sems):
    _sc_body(didx_hbm, aidx_hbm, dtab_hbm, atab_hbm, out_hbm,
             dpidx_v, apidx_v, doff_v, aoff_v, drows_v, arows_v, out_v,
             idx_tmp_v, sems)


@jax.jit
def kernel(drug_idx, adr_idx, drug_embeddings, adr_embeddings):
    dtab = drug_embeddings.reshape(-1, PDIM)
    atab = adr_embeddings.reshape(-1, PDIM)
    return _sc_call(drug_idx, adr_idx, dtab, atab)
